# baseline (speedup 1.0000x reference)
import functools

import jax
import jax.numpy as jnp
from jax import lax
from jax.experimental import pallas as pl
from jax.experimental.pallas import tpu as pltpu
from jax.experimental.pallas import tpu_sc as plsc

VOCAB = 1000
DIM = 64
BATCH = 16384
DIM_PAD = 128

NUM_CORES = 2
NUM_SUBCORES = 16
NW = NUM_CORES * NUM_SUBCORES
B_PER_W = BATCH // NW
IDX_CHUNK = 128
N_CHUNKS = B_PER_W // IDX_CHUNK


def _sc_gather_body(table_hbm, idx_hbm, out_hbm, idx_v, rows_v, sem):
    wid = lax.axis_index("s") * NUM_CORES + lax.axis_index("c")
    base = wid * B_PER_W
    pltpu.sync_copy(idx_hbm.at[pl.ds(wid * N_CHUNKS, N_CHUNKS)], idx_v)
    copies = []
    for j in range(N_CHUNKS):
        copies.append(
            pltpu.async_copy(
                table_hbm.at[idx_v.at[j]],
                rows_v.at[pl.ds(j * IDX_CHUNK, IDX_CHUNK)],
                sem,
            )
        )
    for c in copies:
        c.wait()
    pltpu.sync_copy(rows_v, out_hbm.at[pl.ds(base, B_PER_W)])


def _sc_gather(table, idx2d):
    mesh = plsc.VectorSubcoreMesh(core_axis_name="c", subcore_axis_name="s")
    kern = functools.partial(
        pl.kernel,
        mesh=mesh,
        out_type=jax.ShapeDtypeStruct((BATCH, DIM_PAD), jnp.float32),
        scratch_types=[
            pltpu.VMEM((N_CHUNKS, IDX_CHUNK), jnp.int32),
            pltpu.VMEM((B_PER_W, DIM_PAD), jnp.float32),
            pltpu.SemaphoreType.DMA,
        ],
    )(_sc_gather_body)
    return kern(table, idx2d)


_PROJ_BB = 4096


def _proj_body(w_ref, emb_ref, b_ref, out_ref):
    out_ref[...] = (
        lax.dot_general(
            w_ref[...], emb_ref[...],
            (((1,), (1,)), ((), ())),
            preferred_element_type=jnp.float32,
        )
        + b_ref[...]
    )


def _tc_project_t(w_pad, emb, bcol):
    return pl.pallas_call(
        _proj_body,
        grid=(BATCH // _PROJ_BB,),
        in_specs=[
            pl.BlockSpec((VOCAB, DIM_PAD), lambda i: (0, 0)),
            pl.BlockSpec((_PROJ_BB, DIM_PAD), lambda i: (i, 0)),
            pl.BlockSpec((VOCAB, 1), lambda i: (0, 0)),
        ],
        out_specs=pl.BlockSpec((VOCAB, _PROJ_BB), lambda i: (0, i)),
        out_shape=jax.ShapeDtypeStruct((VOCAB, BATCH), jnp.float32),
    )(w_pad, emb, bcol)


def kernel(inputs, emb_weight, lin_weight, lin_bias):
    idx2d = inputs.astype(jnp.int32).reshape(BATCH // IDX_CHUNK, IDX_CHUNK)
    pad = ((0, 0), (0, DIM_PAD - DIM))
    emb = _sc_gather(jnp.pad(emb_weight, pad), idx2d)
    w_pad = jnp.pad(lin_weight, pad)
    out_t = _tc_project_t(w_pad, emb, lin_bias.reshape(VOCAB, 1))
    return (out_t.T,)


# speedup vs baseline: 1.0117x; 1.0117x over previous
import functools

import jax
import jax.numpy as jnp
from jax import lax
from jax.experimental import pallas as pl
from jax.experimental.pallas import tpu as pltpu
from jax.experimental.pallas import tpu_sc as plsc

VOCAB = 1000
DIM = 64
BATCH = 16384
DIM_PAD = 128

NUM_CORES = 2
NUM_SUBCORES = 16
NW = NUM_CORES * NUM_SUBCORES
B_PER_W = BATCH // NW
IDX_CHUNK = 128
N_CHUNKS = B_PER_W // IDX_CHUNK


def _sc_gather_body(table_hbm, idx_hbm, out_hbm, idx_v, rows_v, gsems, osem):
    wid = lax.axis_index("s") * NUM_CORES + lax.axis_index("c")
    base = wid * B_PER_W
    pltpu.sync_copy(idx_hbm.at[pl.ds(wid * N_CHUNKS, N_CHUNKS)], idx_v)
    gathers = []
    for j in range(N_CHUNKS):
        gathers.append(
            pltpu.async_copy(
                table_hbm.at[idx_v.at[j]],
                rows_v.at[pl.ds(j * IDX_CHUNK, IDX_CHUNK)],
                gsems.at[j],
            )
        )
    outs = []
    for j in range(N_CHUNKS):
        gathers[j].wait()
        outs.append(
            pltpu.async_copy(
                rows_v.at[pl.ds(j * IDX_CHUNK, IDX_CHUNK)],
                out_hbm.at[pl.ds(base + j * IDX_CHUNK, IDX_CHUNK)],
                osem,
            )
        )
    for c in outs:
        c.wait()


def _sc_gather(table, idx2d):
    mesh = plsc.VectorSubcoreMesh(core_axis_name="c", subcore_axis_name="s")
    kern = functools.partial(
        pl.kernel,
        mesh=mesh,
        out_type=jax.ShapeDtypeStruct((BATCH, DIM_PAD), jnp.float32),
        scratch_types=[
            pltpu.VMEM((N_CHUNKS, IDX_CHUNK), jnp.int32),
            pltpu.VMEM((B_PER_W, DIM_PAD), jnp.float32),
            pltpu.SemaphoreType.DMA((N_CHUNKS,)),
            pltpu.SemaphoreType.DMA,
        ],
    )(_sc_gather_body)
    return kern(table, idx2d)


_PROJ_BB = 2048


def _proj_body(w_ref, emb_ref, b_ref, out_ref):
    out_ref[...] = (
        lax.dot_general(
            w_ref[...], emb_ref[...],
            (((1,), (1,)), ((), ())),
            preferred_element_type=jnp.float32,
        )
        + b_ref[...]
    )


def _tc_project_t(w_pad, emb, bcol):
    return pl.pallas_call(
        _proj_body,
        grid=(BATCH // _PROJ_BB,),
        in_specs=[
            pl.BlockSpec((VOCAB, DIM_PAD), lambda i: (0, 0)),
            pl.BlockSpec((_PROJ_BB, DIM_PAD), lambda i: (i, 0)),
            pl.BlockSpec((VOCAB, 1), lambda i: (0, 0)),
        ],
        out_specs=pl.BlockSpec((VOCAB, _PROJ_BB), lambda i: (0, i)),
        out_shape=jax.ShapeDtypeStruct((VOCAB, BATCH), jnp.float32),
    )(w_pad, emb, bcol)


def kernel(inputs, emb_weight, lin_weight, lin_bias):
    idx2d = inputs.astype(jnp.int32).reshape(BATCH // IDX_CHUNK, IDX_CHUNK)
    pad = ((0, 0), (0, DIM_PAD - DIM))
    emb = _sc_gather(jnp.pad(emb_weight, pad), idx2d)
    w_pad = jnp.pad(lin_weight, pad)
    out_t = _tc_project_t(w_pad, emb, lin_bias.reshape(VOCAB, 1))
    return (out_t.T,)
